# Initial kernel scaffold; baseline (speedup 1.0000x reference)
import jax
import jax.numpy as jnp
from jax.experimental import pallas as pl


def kernel(feat, edge_index, W1, b1, W2, b2, W3, b3):
    raise NotImplementedError("write your pallas kernel here")


# speedup vs baseline: 6.2896x; 6.2896x over previous
import functools

import jax
import jax.numpy as jnp
from jax import lax
from jax.experimental import pallas as pl
from jax.experimental.pallas import tpu as pltpu
from jax.experimental.pallas import tpu_sc as plsc

N = 10000
NPAD = 10240
TRASH = 10100
E = 320000
EPAD = 327680
B = 128
ROWS_ES = 80
ROWS_CS = 160
CH = 40
RT = 640
_MESH = plsc.VectorSubcoreMesh(
    core_axis_name="c", subcore_axis_name="s", num_cores=2, num_subcores=16)

f32 = jnp.float32


@functools.partial(
    pl.kernel,
    out_type=jax.ShapeDtypeStruct((2, NPAD, 8), f32),
    mesh=_MESH,
    compiler_params=pltpu.CompilerParams(use_tc_tiling_on_sc=False),
    scratch_types=[
        pltpu.VMEM((B,), jnp.int32),
        pltpu.VMEM((B, 8), f32),
        pltpu.VMEM_SHARED((NPAD, 8), f32),
    ],
)
def _sc_deg(dst2d, zeros8, ones8, out, idx_v, ones_v, acc):
    c = lax.axis_index("c")
    s = lax.axis_index("s")
    pltpu.sync_copy(zeros8.at[pl.ds(s * RT, RT)], acc.at[pl.ds(s * RT, RT)])
    pltpu.sync_copy(ones8, ones_v)
    base = c * (ROWS_ES * 16) + s * ROWS_ES
    plsc.subcore_barrier()

    def body(j, carry):
        pltpu.sync_copy(dst2d.at[base + j], idx_v)
        pltpu.sync_copy(ones_v, acc.at[idx_v], add=True)
        return carry

    lax.fori_loop(0, ROWS_ES, body, 0)
    plsc.subcore_barrier()
    pltpu.sync_copy(acc.at[pl.ds(s * RT, RT)], out.at[c, pl.ds(s * RT, RT)])


def _make_sc_agg_edge(d, tc_tiling=True, rows=ROWS_ES):
    @functools.partial(
        pl.kernel,
        out_type=jax.ShapeDtypeStruct((2, NPAD, d), f32),
        mesh=_MESH,
        compiler_params=pltpu.CompilerParams(use_tc_tiling_on_sc=tc_tiling),
        scratch_types=[
            pltpu.VMEM((CH, B), jnp.int32),
            pltpu.VMEM((CH, B), jnp.int32),
            pltpu.VMEM((B, d), f32),
            pltpu.VMEM_SHARED((NPAD, d), f32),
            pltpu.SemaphoreType.DMA,
        ],
    )
    def agg(table, src2d, dst2d, zeros, out, src_v, dst_v, rows_v, acc, sem):
        c = lax.axis_index("c")
        s = lax.axis_index("s")
        pltpu.sync_copy(zeros.at[pl.ds(s * RT, RT)], acc.at[pl.ds(s * RT, RT)])
        base = c * (rows * 16) + s * rows
        plsc.subcore_barrier()

        def chunk(ch, carry):
            pltpu.sync_copy(src2d.at[pl.ds(base + ch * CH, CH)], src_v)
            pltpu.sync_copy(dst2d.at[pl.ds(base + ch * CH, CH)], dst_v)

            def body(j, carry2):
                pltpu.async_copy(table.at[src_v.at[j]], rows_v, sem).wait()
                pltpu.sync_copy(rows_v, acc.at[dst_v.at[j]], add=True)
                return carry2

            lax.fori_loop(0, CH, body, 0)
            return carry

        lax.fori_loop(0, rows // CH, chunk, 0)
        plsc.subcore_barrier()
        pltpu.sync_copy(acc.at[pl.ds(s * RT, RT)], out.at[c, pl.ds(s * RT, RT)])

    return agg


_sc_agg_e128 = _make_sc_agg_edge(128)
_sc_agg_e40 = _make_sc_agg_edge(40, tc_tiling=False)


@functools.partial(
    pl.kernel,
    out_type=jax.ShapeDtypeStruct((2, NPAD, 128), f32),
    mesh=_MESH,
    scratch_types=[
        pltpu.VMEM((CH, B), jnp.int32),
        pltpu.VMEM((CH, B), jnp.int32),
        pltpu.VMEM((B, 128), f32),
        pltpu.VMEM_SHARED((NPAD, 128), f32),
        pltpu.SemaphoreType.DMA,
    ],
)
def _sc_agg_col(table2, src2d, dst2d, zeros, out, src_v, dst_v, rows_v, acc,
                sem):
    c = lax.axis_index("c")
    s = lax.axis_index("s")
    pltpu.sync_copy(zeros.at[pl.ds(s * RT, RT)], acc.at[pl.ds(s * RT, RT)])
    off = c * N
    plsc.subcore_barrier()

    def chunk(ch, carry):
        pltpu.sync_copy(src2d.at[pl.ds(s * ROWS_CS + ch * CH, CH)], src_v)
        pltpu.sync_copy(dst2d.at[pl.ds(s * ROWS_CS + ch * CH, CH)], dst_v)

        def add_off(i, carry2):
            row = i // (B // 16)
            k = lax.rem(i, B // 16)
            sl = pl.ds(k * 16, 16)
            src_v[row, sl] = src_v[row, sl] + off
            return carry2

        lax.fori_loop(0, CH * (B // 16), add_off, 0)

        def body(j, carry2):
            pltpu.async_copy(table2.at[src_v.at[j]], rows_v, sem).wait()
            pltpu.sync_copy(rows_v, acc.at[dst_v.at[j]], add=True)
            return carry2

        lax.fori_loop(0, CH, body, 0)
        return carry

    lax.fori_loop(0, ROWS_CS // CH, chunk, 0)
    plsc.subcore_barrier()
    pltpu.sync_copy(acc.at[pl.ds(s * RT, RT)], out.at[c, pl.ds(s * RT, RT)])


_GRID = 10
_R = N // _GRID


def _tc_call(fn, out_shapes, in_specs, out_specs):
    return pl.pallas_call(
        fn,
        grid=(_GRID,),
        out_shape=out_shapes,
        in_specs=in_specs,
        out_specs=out_specs,
    )


def _k1_body(degp, feat, d8_o, y_o, z1_o):
    deg = degp[0][:, 0:1] + degp[1][:, 0:1] + 1.0
    d = lax.rsqrt(deg)
    d8_o[...] = jnp.broadcast_to(d, (_R, 8))
    y_o[...] = feat[...] * d
    z1_o[...] = feat[...] * (d * d)


def _k2_body(p1, d8, z1, w1, b1, w2, a_o, b_o, z2_o):
    d = d8[:, 0:1]
    t1 = d * (p1[0] + p1[1]) + z1[...]
    x1 = jnp.maximum(jnp.dot(t1, w1[...], preferred_element_type=f32)
                     + b1[...], 0.0)
    h2 = jnp.dot(x1, w2[...], preferred_element_type=f32)
    hs2 = h2 * d
    a_o[...] = hs2[:, :128]
    b_o[...] = hs2[:, 128:]
    z2_o[...] = h2 * (d * d)


def _k3_body(p2, d8, z2, w3, b2, hs3_o, z3_o):
    d = d8[:, 0:1]
    agg2 = jnp.concatenate([p2[0], p2[1]], axis=1)
    x2 = jnp.maximum(d * agg2 + z2[...] + b2[...], 0.0)
    h3 = jnp.dot(x2, w3[...], preferred_element_type=f32)
    hs3_o[...] = h3 * d
    z3_o[...] = h3 * (d * d)


def _k4_body(p3, d8, z3, b3, probs_o, x3_o):
    d = d8[:, 0:1]
    x3 = d * (p3[0] + p3[1]) + z3[...] + b3[...]
    m = jnp.max(x3, axis=-1, keepdims=True)
    e = jnp.exp(x3 - m)
    probs_o[...] = e / jnp.sum(e, axis=-1, keepdims=True)
    x3_o[...] = x3


def _row_spec(cols):
    return pl.BlockSpec((_R, cols), lambda i: (i, 0))


def _pair_spec(cols):
    return pl.BlockSpec((2, _R, cols), lambda i: (0, i, 0))


def _full_spec(r, c):
    return pl.BlockSpec((r, c), lambda i: (0, 0))


def kernel(feat, edge_index, W1, b1, W2, b2, W3, b3):
    src = edge_index[0]
    dst = edge_index[1]
    src_p = jnp.concatenate(
        [src, jnp.zeros((EPAD - E,), jnp.int32)]).reshape(EPAD // B, B)
    dst_p = jnp.concatenate(
        [dst, jnp.full((EPAD - E,), TRASH, jnp.int32)]).reshape(EPAD // B, B)
    zeros128 = jnp.zeros((NPAD, 128), f32)
    zeros40 = jnp.zeros((NPAD, 40), f32)
    zeros8 = jnp.zeros((NPAD, 8), f32)
    ones8 = jnp.ones((B, 8), f32)

    degp = _sc_deg(dst_p, zeros8, ones8)

    d8, y, z1 = _tc_call(
        _k1_body,
        [jax.ShapeDtypeStruct((N, 8), f32),
         jax.ShapeDtypeStruct((N, 128), f32),
         jax.ShapeDtypeStruct((N, 128), f32)],
        [_pair_spec(8), _row_spec(128)],
        [_row_spec(8), _row_spec(128), _row_spec(128)],
    )(degp, feat)

    p1 = _sc_agg_e128(y, src_p, dst_p, zeros128)

    hs2a, hs2b, z2 = _tc_call(
        _k2_body,
        [jax.ShapeDtypeStruct((N, 128), f32),
         jax.ShapeDtypeStruct((N, 128), f32),
         jax.ShapeDtypeStruct((N, 256), f32)],
        [_pair_spec(128), _row_spec(8), _row_spec(128),
         _full_spec(128, 256), _full_spec(1, 256), _full_spec(256, 256)],
        [_row_spec(128), _row_spec(128), _row_spec(256)],
    )(p1, d8, z1, W1, b1[None, :], W2)

    tab2 = jnp.concatenate([hs2a, hs2b], axis=0)
    p2 = _sc_agg_col(tab2, src_p, dst_p, zeros128)

    hs3, z3 = _tc_call(
        _k3_body,
        [jax.ShapeDtypeStruct((N, 40), f32),
         jax.ShapeDtypeStruct((N, 40), f32)],
        [_pair_spec(128), _row_spec(8), _row_spec(256),
         _full_spec(256, 40), _full_spec(1, 256)],
        [_row_spec(40), _row_spec(40)],
    )(p2, d8, z2, W3, b2[None, :])

    p3 = _sc_agg_e40(hs3, src_p, dst_p, zeros40)

    probs, x3 = _tc_call(
        _k4_body,
        [jax.ShapeDtypeStruct((N, 40), f32),
         jax.ShapeDtypeStruct((N, 40), f32)],
        [_pair_spec(40), _row_spec(8), _row_spec(40), _full_spec(1, 40)],
        [_row_spec(40), _row_spec(40)],
    )(p3, d8, z3, b3[None, :])

    return probs, x3


# speedup vs baseline: 6.8981x; 1.0967x over previous
import functools

import jax
import jax.numpy as jnp
from jax import lax
from jax.experimental import pallas as pl
from jax.experimental.pallas import tpu as pltpu
from jax.experimental.pallas import tpu_sc as plsc

N = 10000
NPAD = 10240
TRASH = 10100
E = 320000
EPAD = 327680
B = 128
ROWS_ES = 80
ROWS_CS = 160
CH = 40
RT = 640
_MESH = plsc.VectorSubcoreMesh(
    core_axis_name="c", subcore_axis_name="s", num_cores=2, num_subcores=16)

f32 = jnp.float32


@functools.partial(
    pl.kernel,
    out_type=jax.ShapeDtypeStruct((2, NPAD, 8), f32),
    mesh=_MESH,
    compiler_params=pltpu.CompilerParams(use_tc_tiling_on_sc=False),
    scratch_types=[
        pltpu.VMEM((B,), jnp.int32),
        pltpu.VMEM((B, 8), f32),
        pltpu.VMEM_SHARED((NPAD, 8), f32),
    ],
)
def _sc_deg(dst2d, zeros8, ones8, out, idx_v, ones_v, acc):
    c = lax.axis_index("c")
    s = lax.axis_index("s")
    pltpu.sync_copy(zeros8.at[pl.ds(s * RT, RT)], acc.at[pl.ds(s * RT, RT)])
    pltpu.sync_copy(ones8, ones_v)
    base = c * (ROWS_ES * 16) + s * ROWS_ES
    plsc.subcore_barrier()

    def body(j, carry):
        pltpu.sync_copy(dst2d.at[base + j], idx_v)
        pltpu.sync_copy(ones_v, acc.at[idx_v], add=True)
        return carry

    lax.fori_loop(0, ROWS_ES, body, 0)
    plsc.subcore_barrier()
    pltpu.sync_copy(acc.at[pl.ds(s * RT, RT)], out.at[c, pl.ds(s * RT, RT)])


def _pipelined_chunk(table, acc, src_v, dst_v, r0, r1, g0, g1, s0, s1):
    def arm(j, x_buf, gx, sx, y_buf, gy, sy):
        pltpu.make_async_copy(table.at[src_v.at[j]], x_buf, gx).wait()

        @pl.when(j + 1 < CH)
        def _():
            @pl.when(j >= 1)
            def _():
                pltpu.make_async_copy(
                    y_buf, acc.at[dst_v.at[j - 1]], sy).wait()

            pltpu.async_copy(table.at[src_v.at[j + 1]], y_buf, gy)

        pltpu.async_copy(x_buf, acc.at[dst_v.at[j]], sx, add=True)

    pltpu.async_copy(table.at[src_v.at[0]], r0, g0)

    def step(j, carry):
        @pl.when(lax.rem(j, 2) == 0)
        def _():
            arm(j, r0, g0, s0, r1, g1, s1)

        @pl.when(lax.rem(j, 2) == 1)
        def _():
            arm(j, r1, g1, s1, r0, g0, s0)

        return carry

    lax.fori_loop(0, CH, step, 0)
    last = CH - 1
    if last % 2 == 0:
        pltpu.make_async_copy(r0, acc.at[dst_v.at[last]], s0).wait()
    else:
        pltpu.make_async_copy(r1, acc.at[dst_v.at[last]], s1).wait()


def _make_sc_agg_edge(d, tc_tiling=True, rows=ROWS_ES):
    @functools.partial(
        pl.kernel,
        out_type=jax.ShapeDtypeStruct((2, NPAD, d), f32),
        mesh=_MESH,
        compiler_params=pltpu.CompilerParams(use_tc_tiling_on_sc=tc_tiling),
        scratch_types=[
            pltpu.VMEM((CH, B), jnp.int32),
            pltpu.VMEM((CH, B), jnp.int32),
            pltpu.VMEM((B, d), f32),
            pltpu.VMEM((B, d), f32),
            pltpu.VMEM_SHARED((NPAD, d), f32),
            pltpu.SemaphoreType.DMA,
            pltpu.SemaphoreType.DMA,
            pltpu.SemaphoreType.DMA,
            pltpu.SemaphoreType.DMA,
        ],
    )
    def agg(table, src2d, dst2d, zeros, out, src_v, dst_v, r0, r1, acc,
            g0, g1, s0, s1):
        c = lax.axis_index("c")
        s = lax.axis_index("s")
        pltpu.sync_copy(zeros.at[pl.ds(s * RT, RT)], acc.at[pl.ds(s * RT, RT)])
        base = c * (rows * 16) + s * rows
        plsc.subcore_barrier()

        def chunk(ch, carry):
            pltpu.sync_copy(src2d.at[pl.ds(base + ch * CH, CH)], src_v)
            pltpu.sync_copy(dst2d.at[pl.ds(base + ch * CH, CH)], dst_v)
            _pipelined_chunk(table, acc, src_v, dst_v, r0, r1, g0, g1, s0, s1)
            return carry

        lax.fori_loop(0, rows // CH, chunk, 0)
        plsc.subcore_barrier()
        pltpu.sync_copy(acc.at[pl.ds(s * RT, RT)], out.at[c, pl.ds(s * RT, RT)])

    return agg


_sc_agg_e128 = _make_sc_agg_edge(128)
_sc_agg_e40 = _make_sc_agg_edge(40, tc_tiling=False)


@functools.partial(
    pl.kernel,
    out_type=jax.ShapeDtypeStruct((2, NPAD, 128), f32),
    mesh=_MESH,
    scratch_types=[
        pltpu.VMEM((CH, B), jnp.int32),
        pltpu.VMEM((CH, B), jnp.int32),
        pltpu.VMEM((B, 128), f32),
        pltpu.VMEM((B, 128), f32),
        pltpu.VMEM_SHARED((NPAD, 128), f32),
        pltpu.SemaphoreType.DMA,
        pltpu.SemaphoreType.DMA,
        pltpu.SemaphoreType.DMA,
        pltpu.SemaphoreType.DMA,
    ],
)
def _sc_agg_col(table2, src2d, dst2d, zeros, out, src_v, dst_v, r0, r1, acc,
                g0, g1, s0, s1):
    c = lax.axis_index("c")
    s = lax.axis_index("s")
    pltpu.sync_copy(zeros.at[pl.ds(s * RT, RT)], acc.at[pl.ds(s * RT, RT)])
    off = c * N
    plsc.subcore_barrier()

    def chunk(ch, carry):
        pltpu.sync_copy(src2d.at[pl.ds(s * ROWS_CS + ch * CH, CH)], src_v)
        pltpu.sync_copy(dst2d.at[pl.ds(s * ROWS_CS + ch * CH, CH)], dst_v)

        def add_off(i, carry2):
            row = i // (B // 16)
            k = lax.rem(i, B // 16)
            sl = pl.ds(k * 16, 16)
            src_v[row, sl] = src_v[row, sl] + off
            return carry2

        lax.fori_loop(0, CH * (B // 16), add_off, 0)
        _pipelined_chunk(table2, acc, src_v, dst_v, r0, r1, g0, g1, s0, s1)
        return carry

    lax.fori_loop(0, ROWS_CS // CH, chunk, 0)
    plsc.subcore_barrier()
    pltpu.sync_copy(acc.at[pl.ds(s * RT, RT)], out.at[c, pl.ds(s * RT, RT)])


_GRID = 10
_R = N // _GRID


def _tc_call(fn, out_shapes, in_specs, out_specs):
    return pl.pallas_call(
        fn,
        grid=(_GRID,),
        out_shape=out_shapes,
        in_specs=in_specs,
        out_specs=out_specs,
    )


def _k1_body(degp, feat, d8_o, y_o, z1_o):
    deg = degp[0][:, 0:1] + degp[1][:, 0:1] + 1.0
    d = lax.rsqrt(deg)
    d8_o[...] = jnp.broadcast_to(d, (_R, 8))
    y_o[...] = feat[...] * d
    z1_o[...] = feat[...] * (d * d)


def _k2_body(p1, d8, z1, w1, b1, w2, a_o, b_o, z2_o):
    d = d8[:, 0:1]
    t1 = d * (p1[0] + p1[1]) + z1[...]
    x1 = jnp.maximum(jnp.dot(t1, w1[...], preferred_element_type=f32)
                     + b1[...], 0.0)
    h2 = jnp.dot(x1, w2[...], preferred_element_type=f32)
    hs2 = h2 * d
    a_o[...] = hs2[:, :128]
    b_o[...] = hs2[:, 128:]
    z2_o[...] = h2 * (d * d)


def _k3_body(p2, d8, z2, w3, b2, hs3_o, z3_o):
    d = d8[:, 0:1]
    agg2 = jnp.concatenate([p2[0], p2[1]], axis=1)
    x2 = jnp.maximum(d * agg2 + z2[...] + b2[...], 0.0)
    h3 = jnp.dot(x2, w3[...], preferred_element_type=f32)
    hs3_o[...] = h3 * d
    z3_o[...] = h3 * (d * d)


def _k4_body(p3, d8, z3, b3, probs_o, x3_o):
    d = d8[:, 0:1]
    x3 = d * (p3[0] + p3[1]) + z3[...] + b3[...]
    m = jnp.max(x3, axis=-1, keepdims=True)
    e = jnp.exp(x3 - m)
    probs_o[...] = e / jnp.sum(e, axis=-1, keepdims=True)
    x3_o[...] = x3


def _row_spec(cols):
    return pl.BlockSpec((_R, cols), lambda i: (i, 0))


def _pair_spec(cols):
    return pl.BlockSpec((2, _R, cols), lambda i: (0, i, 0))


def _full_spec(r, c):
    return pl.BlockSpec((r, c), lambda i: (0, 0))


def kernel(feat, edge_index, W1, b1, W2, b2, W3, b3):
    src = edge_index[0]
    dst = edge_index[1]
    src_p = jnp.concatenate(
        [src, jnp.zeros((EPAD - E,), jnp.int32)]).reshape(EPAD // B, B)
    dst_p = jnp.concatenate(
        [dst, jnp.full((EPAD - E,), TRASH, jnp.int32)]).reshape(EPAD // B, B)
    zeros128 = jnp.zeros((NPAD, 128), f32)
    zeros40 = jnp.zeros((NPAD, 40), f32)
    zeros8 = jnp.zeros((NPAD, 8), f32)
    ones8 = jnp.ones((B, 8), f32)

    degp = _sc_deg(dst_p, zeros8, ones8)

    d8, y, z1 = _tc_call(
        _k1_body,
        [jax.ShapeDtypeStruct((N, 8), f32),
         jax.ShapeDtypeStruct((N, 128), f32),
         jax.ShapeDtypeStruct((N, 128), f32)],
        [_pair_spec(8), _row_spec(128)],
        [_row_spec(8), _row_spec(128), _row_spec(128)],
    )(degp, feat)

    p1 = _sc_agg_e128(y, src_p, dst_p, zeros128)

    hs2a, hs2b, z2 = _tc_call(
        _k2_body,
        [jax.ShapeDtypeStruct((N, 128), f32),
         jax.ShapeDtypeStruct((N, 128), f32),
         jax.ShapeDtypeStruct((N, 256), f32)],
        [_pair_spec(128), _row_spec(8), _row_spec(128),
         _full_spec(128, 256), _full_spec(1, 256), _full_spec(256, 256)],
        [_row_spec(128), _row_spec(128), _row_spec(256)],
    )(p1, d8, z1, W1, b1[None, :], W2)

    tab2 = jnp.concatenate([hs2a, hs2b], axis=0)
    p2 = _sc_agg_col(tab2, src_p, dst_p, zeros128)

    hs3, z3 = _tc_call(
        _k3_body,
        [jax.ShapeDtypeStruct((N, 40), f32),
         jax.ShapeDtypeStruct((N, 40), f32)],
        [_pair_spec(128), _row_spec(8), _row_spec(256),
         _full_spec(256, 40), _full_spec(1, 256)],
        [_row_spec(40), _row_spec(40)],
    )(p2, d8, z2, W3, b2[None, :])

    p3 = _sc_agg_e40(hs3, src_p, dst_p, zeros40)

    probs, x3 = _tc_call(
        _k4_body,
        [jax.ShapeDtypeStruct((N, 40), f32),
         jax.ShapeDtypeStruct((N, 40), f32)],
        [_pair_spec(40), _row_spec(8), _row_spec(40), _full_spec(1, 40)],
        [_row_spec(40), _row_spec(40)],
    )(p3, d8, z3, b3[None, :])

    return probs, x3
